# baseline (speedup 1.0000x reference)
import functools

import jax
import jax.numpy as jnp
from jax import lax
from jax.experimental import pallas as pl
from jax.experimental.pallas import tpu as pltpu
from jax.experimental.pallas import tpu_sc as plsc

NUM_FIELDS = 26
VOCAB = 100000
BATCH = 16384
DIM = 64
LANES = 16

NUM_CORES = 2
NUM_SUBCORES = 16
NUM_WORKERS = NUM_CORES * NUM_SUBCORES
BPW = BATCH // NUM_WORKERS

TCOLS = 782
QR = TCOLS * 32
NFH = NUM_FIELDS // 2

_MESH = plsc.VectorSubcoreMesh(core_axis_name="c", subcore_axis_name="s")
_CPARAMS = pltpu.CompilerParams(use_tc_tiling_on_sc=True, needs_layout_passes=False)


def _repack_body(*refs):
    wts = refs[:NFH]
    P = refs[NFH]
    blk = refs[NFH + 1 : NFH + 5]
    blk32 = refs[NFH + 5]
    ob = refs[NFH + 6 : NFH + 10]
    sin = refs[NFH + 10 : NFH + 14]
    sout = refs[NFH + 14 : NFH + 18]

    wid = lax.axis_index("c") * NUM_SUBCORES + lax.axis_index("s")

    def start_in(wt, tc, b):
        pltpu.async_copy(wt.at[:, pl.ds(tc * 128, 128)], blk[b], sin[b])

    def wait_in(wt, tc, b):
        pltpu.make_async_copy(
            wt.at[:, pl.ds(tc * 128, 128)], blk[b], sin[b]
        ).wait()

    iota = lax.iota(jnp.int32, LANES)
    qoff = lax.shift_right_logical(iota, 2)
    woff = (iota & 3) << 5

    def compute(b):
        @plsc.parallel_loop(0, 256, unroll=4)
        def tbody(t):
            c = t >> 3
            rv0 = (t & 7) * 16
            a = plsc.bitcast(blk[b][c, pl.ds(rv0, 16)], jnp.int32)
            bb = plsc.bitcast(blk[b][c + 32, pl.ds(rv0, 16)], jnp.int32)
            w = lax.shift_right_logical(a, 16) | (bb & jnp.int32(-65536))
            plsc.store_scatter(ob[b], [(rv0 >> 2) + qoff, woff + c], w)

    def start_out(f, tc, b):
        pltpu.async_copy(ob[b], P.at[pl.ds(f * QR + tc * 32, 32)], sout[b])

    def wait_out(f, b):
        pltpu.make_async_copy(ob[b], P.at[pl.ds(f * QR, 32)], sout[b]).wait()

    for f in range(NFH):
        wt = wts[f]
        nk = (TCOLS - 2 - wid) // 32 + 1
        for b0 in range(4):
            start_in(wt, wid + 32 * b0, b0)

        def quad(kk, carry, wt=wt, f=f):
            for b in range(4):
                k = kk * 4 + b

                @pl.when(k < nk)
                def _(k=k, b=b):
                    tc = wid + 32 * k
                    wait_in(wt, tc, b)

                    @pl.when(kk > 0)
                    def _():
                        wait_out(f, b)

                    compute(b)
                    start_out(f, tc, b)

                    @pl.when(k + 4 < nk)
                    def _():
                        start_in(wt, tc + 128, b)

            return carry

        lax.fori_loop(0, (nk + 3) // 4, quad, 0)
        for b0 in range(4):
            wait_out(f, b0)

    iota = lax.iota(jnp.int32, LANES)
    for f in range(NFH):

        @pl.when(wid == f)
        def _(f=f):
            pltpu.sync_copy(wts[f].at[:, pl.ds((TCOLS - 1) * 128, 32)], blk32)

            def qtail(t, carry):
                q = t >> 3
                wq = t & 7
                p = wq >> 1
                cvec = (wq & 1) * 16 + iota
                rvv = jnp.full((LANES,), 0, jnp.int32) + (q * 4 + p)
                a = plsc.load_gather(blk32, [cvec, rvv])
                bb = plsc.load_gather(blk32, [cvec + 32, rvv])
                w = lax.shift_right_logical(
                    plsc.bitcast(a, jnp.int32), 16
                ) | (plsc.bitcast(bb, jnp.int32) & jnp.int32(-65536))
                ob[0][q, pl.ds(wq * 16, 16)] = w
                return carry

            lax.fori_loop(0, 64, qtail, 0)
            pltpu.sync_copy(
                ob[0].at[pl.ds(0, 8)],
                P.at[pl.ds(f * QR + (TCOLS - 1) * 32, 8)],
            )


_repack = functools.partial(
    pl.kernel,
    mesh=_MESH,
    compiler_params=_CPARAMS,
    out_type=jax.ShapeDtypeStruct((NFH * QR, 128), jnp.int32),
    scratch_types=[
        pltpu.VMEM((64, 128), jnp.float32),
        pltpu.VMEM((64, 128), jnp.float32),
        pltpu.VMEM((64, 128), jnp.float32),
        pltpu.VMEM((64, 128), jnp.float32),
        pltpu.VMEM((64, 32), jnp.float32),
        pltpu.VMEM((32, 128), jnp.int32),
        pltpu.VMEM((32, 128), jnp.int32),
        pltpu.VMEM((32, 128), jnp.int32),
        pltpu.VMEM((32, 128), jnp.int32),
        pltpu.SemaphoreType.DMA,
        pltpu.SemaphoreType.DMA,
        pltpu.SemaphoreType.DMA,
        pltpu.SemaphoreType.DMA,
        pltpu.SemaphoreType.DMA,
        pltpu.SemaphoreType.DMA,
        pltpu.SemaphoreType.DMA,
        pltpu.SemaphoreType.DMA,
    ],
)(_repack_body)


def _gather_body(*refs):
    cats = refs[:NUM_FIELDS]
    PA = refs[NUM_FIELDS]
    PB = refs[NUM_FIELDS + 1]
    out = refs[NUM_FIELDS + 2]
    idxo, idxq, acct, g0, g1, sem_idx, sg0, sg1 = refs[NUM_FIELDS + 3 :]

    wid = lax.axis_index("c") * NUM_SUBCORES + lax.axis_index("s")
    base = wid * BPW
    iota = lax.iota(jnp.int32, LANES)

    pend = []
    for f in range(NUM_FIELDS):
        for j in range(4):
            d = pltpu.async_copy(
                cats[f].at[pl.ds(base + j * 128, 128)],
                idxo.at[f * 4 + j],
                sem_idx,
            )
            pend.append(d)
            if len(pend) == 8:
                for d2 in pend:
                    d2.wait()
                pend = []
    for d2 in pend:
        d2.wait()

    for f in range(NUM_FIELDS):

        def tb(j, carry, f=f):
            for h in range(8):
                sl = pl.ds(h * 16, 16)
                v = idxo[f * 4 + j, sl]
                idxq[f * 4 + j, sl] = (v >> 2) + (f % NFH) * QR
            return carry

        lax.fori_loop(0, 4, tb, 0)

    zeros = jnp.zeros((LANES,), jnp.float32)

    def zb(z, carry):
        for c16 in range(BPW // 16):
            acct[z, pl.ds(c16 * 16, 16)] = zeros
        return carry

    lax.fori_loop(0, DIM, zb, 0)

    gb = (g0, g1)
    sg = (sg0, sg1)

    def issue_half(f, h, b):
        P = PA if f < NFH else PB
        for j2 in range(2):
            j = 2 * h + j2
            pltpu.async_copy(
                P.at[idxq.at[f * 4 + j]], gb[b].at[pl.ds(j2 * 128, 128)], sg[b]
            )

    def wait_half(f, h, b):
        P = PA if f < NFH else PB
        for j2 in range(2):
            j = 2 * h + j2
            pltpu.make_async_copy(
                P.at[idxq.at[f * 4 + j]], gb[b].at[pl.ds(j2 * 128, 128)], sg[b]
            ).wait()

    def acc_half(f, h, b):
        def kb(k, carry):
            jrow = f * 4 + h * 2 + (k >> 3)
            och = idxo[jrow, pl.ds((k & 7) * 16, 16)]
            off = (och & 3) << 5
            rows = k * 16 + iota
            csl = pl.ds(h * 256 + k * 16, 16)

            @plsc.parallel_loop(0, 8, unroll=2)
            def mb(mi):
                for dm in range(4):
                    m = mi * 4 + dm
                    wv = plsc.load_gather(gb[b], [rows, off + m])
                    lo = plsc.bitcast(wv << 16, jnp.float32)
                    hi = plsc.bitcast(wv & jnp.int32(-65536), jnp.float32)
                    plsc.addupdate(acct.at[m, csl], lo)
                    plsc.addupdate(acct.at[m + 32, csl], hi)

            return carry

        lax.fori_loop(0, 16, kb, 0)

    issue_half(0, 0, 0)
    issue_half(0, 1, 1)
    for f in range(NUM_FIELDS):
        wait_half(f, 0, 0)
        acc_half(f, 0, 0)
        if f + 1 < NUM_FIELDS:
            issue_half(f + 1, 0, 0)
        wait_half(f, 1, 1)
        acc_half(f, 1, 1)
        if f + 1 < NUM_FIELDS:
            issue_half(f + 1, 1, 1)

    pltpu.sync_copy(acct, out.at[:, pl.ds(base, BPW)])


_gather = functools.partial(
    pl.kernel,
    mesh=_MESH,
    compiler_params=_CPARAMS,
    out_type=jax.ShapeDtypeStruct((DIM, BATCH), jnp.float32),
    scratch_types=[
        pltpu.VMEM((NUM_FIELDS * 4, 128), jnp.int32),
        pltpu.VMEM((NUM_FIELDS * 4, 128), jnp.int32),
        pltpu.VMEM((DIM, BPW), jnp.float32),
        pltpu.VMEM((256, 128), jnp.int32),
        pltpu.VMEM((256, 128), jnp.int32),
        pltpu.SemaphoreType.DMA,
        pltpu.SemaphoreType.DMA,
        pltpu.SemaphoreType.DMA,
    ],
)(_gather_body)


def kernel(cat_0, W_0, cat_1, W_1, cat_2, W_2, cat_3, W_3, cat_4, W_4, cat_5, W_5, cat_6, W_6, cat_7, W_7, cat_8, W_8, cat_9, W_9, cat_10, W_10, cat_11, W_11, cat_12, W_12, cat_13, W_13, cat_14, W_14, cat_15, W_15, cat_16, W_16, cat_17, W_17, cat_18, W_18, cat_19, W_19, cat_20, W_20, cat_21, W_21, cat_22, W_22, cat_23, W_23, cat_24, W_24, cat_25, W_25):
    args = locals()
    wts = [args[f"W_{i}"].T for i in range(NUM_FIELDS)]
    cats = [args[f"cat_{i}"] for i in range(NUM_FIELDS)]
    PA = _repack(*wts[:NFH])
    PB = _repack(*wts[NFH:])
    out_t = _gather(*cats, PA, PB)
    return out_t.T


# speedup vs baseline: 1.3105x; 1.3105x over previous
import functools

import jax
import jax.numpy as jnp
from jax import lax
from jax.experimental import pallas as pl
from jax.experimental.pallas import tpu as pltpu
from jax.experimental.pallas import tpu_sc as plsc

NUM_FIELDS = 26
VOCAB = 100000
BATCH = 16384
DIM = 64
LANES = 16

NUM_CORES = 2
NUM_SUBCORES = 16
NUM_WORKERS = NUM_CORES * NUM_SUBCORES
BPW = BATCH // NUM_WORKERS
CHUNK = 128
NCH = BPW // CHUNK
ROW_UNROLL = 8


def _accumulate(acc, buf):
    def body(r, carry):
        for dr in range(ROW_UNROLL):
            row = r * ROW_UNROLL + dr
            for c in range(DIM // LANES):
                sl = pl.ds(c * LANES, LANES)
                plsc.addupdate(acc.at[row, sl], buf[row, sl])
        return carry

    lax.fori_loop(0, BPW // ROW_UNROLL, body, 0)


def _body(*refs):
    ins = refs[: 2 * NUM_FIELDS]
    out = refs[2 * NUM_FIELDS]
    idx, acc, buf_a, buf_b, sem_idx, sem_a, sem_b, sem_acc = refs[2 * NUM_FIELDS + 1 :]
    cats = ins[0::2]
    tables = ins[1::2]

    wid = lax.axis_index("c") * NUM_SUBCORES + lax.axis_index("s")
    base = wid * BPW

    pend = []
    for i in range(NUM_FIELDS):
        for j in range(NCH):
            d = pltpu.async_copy(
                cats[i].at[pl.ds(base + j * CHUNK, CHUNK)],
                idx.at[i * NCH + j],
                sem_idx,
            )
            pend.append(d)
            if len(pend) == 8:
                for d2 in pend:
                    d2.wait()
                pend = []
    for d2 in pend:
        d2.wait()

    def start_field(i, dst, sem):
        descs = []
        for j in range(NCH):
            idx_view = idx.at[i * NCH + j]
            descs.append(
                pltpu.async_copy(
                    tables[i].at[idx_view],
                    dst.at[pl.ds(j * CHUNK, CHUNK)],
                    sem,
                )
            )
        return descs

    bufs = (buf_a, buf_b)
    sems = (sem_a, sem_b)

    d_acc = start_field(0, acc, sem_acc)
    inflight = [start_field(1, buf_a, sem_a), None]
    for d in d_acc:
        d.wait()
    inflight[1] = start_field(2, buf_b, sem_b)

    for i in range(1, NUM_FIELDS):
        b = (i - 1) % 2
        for d in inflight[b]:
            d.wait()
        _accumulate(acc, bufs[b])
        nxt = i + 2
        if nxt < NUM_FIELDS:
            inflight[b] = start_field(nxt, bufs[b], sems[b])

    pltpu.sync_copy(acc, out.at[pl.ds(base, BPW)])


@functools.partial(
    pl.kernel,
    mesh=plsc.VectorSubcoreMesh(core_axis_name="c", subcore_axis_name="s"),
    compiler_params=pltpu.CompilerParams(use_tc_tiling_on_sc=False),
    out_type=jax.ShapeDtypeStruct((BATCH, DIM), jnp.float32),
    scratch_types=[
        pltpu.VMEM((NUM_FIELDS * NCH, CHUNK), jnp.int32),
        pltpu.VMEM((BPW, DIM), jnp.float32),
        pltpu.VMEM((BPW, DIM), jnp.float32),
        pltpu.VMEM((BPW, DIM), jnp.float32),
        pltpu.SemaphoreType.DMA,
        pltpu.SemaphoreType.DMA,
        pltpu.SemaphoreType.DMA,
        pltpu.SemaphoreType.DMA,
    ],
)
def _embed_sum(*refs):
    _body(*refs)


def kernel(cat_0, W_0, cat_1, W_1, cat_2, W_2, cat_3, W_3, cat_4, W_4, cat_5, W_5, cat_6, W_6, cat_7, W_7, cat_8, W_8, cat_9, W_9, cat_10, W_10, cat_11, W_11, cat_12, W_12, cat_13, W_13, cat_14, W_14, cat_15, W_15, cat_16, W_16, cat_17, W_17, cat_18, W_18, cat_19, W_19, cat_20, W_20, cat_21, W_21, cat_22, W_22, cat_23, W_23, cat_24, W_24, cat_25, W_25):
    args = locals()
    flat = []
    for i in range(NUM_FIELDS):
        flat.append(args[f"cat_{i}"])
        flat.append(args[f"W_{i}"])
    return _embed_sum(*flat)
